# baseline (speedup 1.0000x reference)
import functools

import jax
import jax.numpy as jnp
from jax import lax
from jax.experimental import pallas as pl
from jax.experimental.pallas import tpu as pltpu
from jax.experimental.pallas import tpu_sc as plsc

_N_CODES = 512
_CODE_DIM = 32
_COMMITMENT = 0.25
_ROWS = 64 * 1024
_BLOCK = 1024
_GRID = _ROWS // _BLOCK

_NW = 32
_B_PER_W = _ROWS // _NW
_CHUNK = 128
_NCHUNK = _B_PER_W // _CHUNK


def _vq_body(z_ref, e_ref, zsq_ref, esq_ref, idx_ref, loss_ref):
    i = pl.program_id(0)
    z = z_ref[...]
    e = e_ref[...]
    scores = jax.lax.dot_general(
        z.astype(jnp.bfloat16), e.astype(jnp.bfloat16), (((1,), (1,)), ((), ())),
        preferred_element_type=jnp.float32)
    base = zsq_ref[...] + esq_ref[...]
    dist = base - 2.0 * scores
    m = jnp.min(dist, axis=1, keepdims=True)
    mask = dist == m
    iota = jax.lax.broadcasted_iota(jnp.int32, (_BLOCK, _N_CODES), 1)
    idx = jnp.min(jnp.where(mask, iota, _N_CODES), axis=1).astype(jnp.int32)
    idx_ref[0, 0, :] = idx

    @pl.when(i == 0)
    def _init():
        loss_ref[...] = jnp.zeros_like(loss_ref)

    loss_ref[...] += jnp.sum(jnp.where(mask, dist, 0.0), axis=0, keepdims=True)


_GRP = 16
_NGRP = _B_PER_W // _GRP


@functools.partial(
    pl.kernel,
    mesh=plsc.VectorSubcoreMesh(core_axis_name="c", subcore_axis_name="s"),
    compiler_params=pltpu.CompilerParams(needs_layout_passes=False),
    out_type=jax.ShapeDtypeStruct((_ROWS * _CODE_DIM,), jnp.float32),
    scratch_types=[
        pltpu.VMEM((_N_CODES * _CODE_DIM,), jnp.float32),
        pltpu.VMEM((_NGRP, _GRP), jnp.int32),
        pltpu.VMEM((_B_PER_W * _CODE_DIM,), jnp.float32),
    ],
)
def _sc_gather(table_hbm, idx_hbm, out_hbm, table_v, idx_v, out_v):
    wid = lax.axis_index("s") * 2 + lax.axis_index("c")
    pltpu.sync_copy(table_hbm, table_v)
    pltpu.sync_copy(idx_hbm.at[wid], idx_v)
    lane = lax.iota(jnp.int32, _GRP)

    def body(g, _):
        iv = idx_v[g] * _CODE_DIM
        rowpos = (g * _GRP + lane) * _CODE_DIM
        for dd in range(_CODE_DIM):
            vals = plsc.load_gather(table_v, [iv + dd])
            plsc.store_scatter(out_v, [rowpos + dd], vals)
        return _

    lax.fori_loop(0, _NGRP, body, None)
    pltpu.sync_copy(out_v, out_hbm.at[pl.ds(wid * _B_PER_W * _CODE_DIM,
                                            _B_PER_W * _CODE_DIM)])


@jax.jit
def _vq(zf, embedding, zsq, esq):
    idx, loss = pl.pallas_call(
        _vq_body,
        grid=(_GRID,),
        in_specs=[
            pl.BlockSpec((_BLOCK, _CODE_DIM), lambda i: (i, 0)),
            pl.BlockSpec((_N_CODES, _CODE_DIM), lambda i: (0, 0)),
            pl.BlockSpec((_BLOCK, 1), lambda i: (i, 0)),
            pl.BlockSpec((1, _N_CODES), lambda i: (0, 0)),
        ],
        out_specs=[
            pl.BlockSpec((1, 1, _BLOCK), lambda i: (i, 0, 0)),
            pl.BlockSpec((1, _N_CODES), lambda i: (0, 0)),
        ],
        out_shape=[
            jax.ShapeDtypeStruct((_GRID, 1, _BLOCK), jnp.int32),
            jax.ShapeDtypeStruct((1, _N_CODES), jnp.float32),
        ],
    )(zf, embedding, zsq, esq)
    return idx, loss


def kernel(z, embedding):
    b, n, d = z.shape
    zf = z.reshape(b * n, d)
    zsq = jnp.sum(zf ** 2, axis=-1, keepdims=True)
    esq = jnp.sum(embedding ** 2, axis=-1)[None, :]
    idx, loss = _vq(zf, embedding, zsq, esq)
    idx3 = idx.reshape(_NW, _NGRP, _GRP)
    zq = _sc_gather(embedding.reshape(-1), idx3).reshape(_ROWS, _CODE_DIM)
    vq_loss = jnp.sum(loss) * ((1.0 + _COMMITMENT) / (b * n * d))
    return zq.reshape(b, n, d), idx.reshape(b, n), vq_loss


# speedup vs baseline: 1.2536x; 1.2536x over previous
import jax
import jax.numpy as jnp
from jax.experimental import pallas as pl

_N_CODES = 512
_CODE_DIM = 32
_COMMITMENT = 0.25
_ROWS = 64 * 1024
_BLOCK = 1024
_GRID = _ROWS // _BLOCK


def _vq_body(z_ref, e_ref, ehi_ref, elo_ref, zsq_ref, esq_ref,
             zq_ref, idx_ref, loss_ref):
    i = pl.program_id(0)
    z = z_ref[...]
    e = e_ref[...]
    scores = jax.lax.dot_general(
        z.astype(jnp.bfloat16), e.astype(jnp.bfloat16), (((1,), (1,)), ((), ())),
        preferred_element_type=jnp.float32)
    base = zsq_ref[...] + esq_ref[...]
    dist = base - 2.0 * scores
    m = jnp.min(dist, axis=1, keepdims=True)
    iota = jax.lax.broadcasted_iota(jnp.int32, (_BLOCK, _N_CODES), 1)
    idx = jnp.min(jnp.where(dist == m, iota, _N_CODES), axis=1).astype(jnp.int32)
    idx_ref[0, 0, :] = idx
    onehot = (iota == idx[:, None]).astype(jnp.bfloat16)
    zq = (jax.lax.dot_general(
              onehot, ehi_ref[...], (((1,), (0,)), ((), ())),
              preferred_element_type=jnp.float32)
          + jax.lax.dot_general(
              onehot, elo_ref[...], (((1,), (0,)), ((), ())),
              preferred_element_type=jnp.float32))
    zq_ref[...] = z + (zq - z)
    diff = zq - z

    @pl.when(i == 0)
    def _init():
        loss_ref[...] = jnp.zeros_like(loss_ref)

    loss_ref[...] += jnp.sum(diff * diff, axis=0, keepdims=True)


@jax.jit
def _vq(zf, embedding, ehi, elo, zsq, esq):
    zq, idx, loss = pl.pallas_call(
        _vq_body,
        grid=(_GRID,),
        in_specs=[
            pl.BlockSpec((_BLOCK, _CODE_DIM), lambda i: (i, 0)),
            pl.BlockSpec((_N_CODES, _CODE_DIM), lambda i: (0, 0)),
            pl.BlockSpec((_N_CODES, _CODE_DIM), lambda i: (0, 0)),
            pl.BlockSpec((_N_CODES, _CODE_DIM), lambda i: (0, 0)),
            pl.BlockSpec((_BLOCK, 1), lambda i: (i, 0)),
            pl.BlockSpec((1, _N_CODES), lambda i: (0, 0)),
        ],
        out_specs=[
            pl.BlockSpec((_BLOCK, _CODE_DIM), lambda i: (i, 0)),
            pl.BlockSpec((1, 1, _BLOCK), lambda i: (i, 0, 0)),
            pl.BlockSpec((1, _CODE_DIM), lambda i: (0, 0)),
        ],
        out_shape=[
            jax.ShapeDtypeStruct((_ROWS, _CODE_DIM), jnp.float32),
            jax.ShapeDtypeStruct((_GRID, 1, _BLOCK), jnp.int32),
            jax.ShapeDtypeStruct((1, _CODE_DIM), jnp.float32),
        ],
    )(zf, embedding, ehi, elo, zsq, esq)
    return zq, idx, loss


def kernel(z, embedding):
    b, n, d = z.shape
    zf = z.reshape(b * n, d)
    zsq = jnp.sum(zf ** 2, axis=-1, keepdims=True)
    esq = jnp.sum(embedding ** 2, axis=-1)[None, :]
    ehi = embedding.astype(jnp.bfloat16)
    elo = (embedding - ehi.astype(jnp.float32)).astype(jnp.bfloat16)
    zq, idx, loss = _vq(zf, embedding, ehi, elo, zsq, esq)
    vq_loss = jnp.sum(loss) * ((1.0 + _COMMITMENT) / (b * n * d))
    return zq.reshape(b, n, d), idx.reshape(b, n), vq_loss


# speedup vs baseline: 1.3265x; 1.0581x over previous
import jax
import jax.numpy as jnp
from jax.experimental import pallas as pl

_N_CODES = 512
_CODE_DIM = 32
_COMMITMENT = 0.25
_ROWS = 64 * 1024
_BLOCK = 2048
_GRID = _ROWS // _BLOCK


def _vq_body(z_ref, e_ref, ehi_ref, elo_ref, zsq_ref, esq_ref,
             zq_ref, idx_ref, loss_ref):
    i = pl.program_id(0)
    z = z_ref[...]
    e = e_ref[...]
    scores = jax.lax.dot_general(
        z.astype(jnp.bfloat16), e.astype(jnp.bfloat16), (((1,), (1,)), ((), ())),
        preferred_element_type=jnp.float32)
    base = zsq_ref[...] + esq_ref[...]
    dist = base - 2.0 * scores
    m = jnp.min(dist, axis=1, keepdims=True)
    iota = jax.lax.broadcasted_iota(jnp.int32, (_BLOCK, _N_CODES), 1)
    idx = jnp.min(jnp.where(dist == m, iota, _N_CODES), axis=1).astype(jnp.int32)
    idx_ref[0, 0, :] = idx
    onehot = (iota == idx[:, None]).astype(jnp.bfloat16)
    zq = (jax.lax.dot_general(
              onehot, ehi_ref[...], (((1,), (0,)), ((), ())),
              preferred_element_type=jnp.float32)
          + jax.lax.dot_general(
              onehot, elo_ref[...], (((1,), (0,)), ((), ())),
              preferred_element_type=jnp.float32))
    zq_ref[...] = z + (zq - z)
    diff = zq - z

    @pl.when(i == 0)
    def _init():
        loss_ref[...] = jnp.zeros_like(loss_ref)

    loss_ref[...] += jnp.sum(diff * diff, axis=0, keepdims=True)


@jax.jit
def _vq(zf, embedding, ehi, elo, zsq, esq):
    zq, idx, loss = pl.pallas_call(
        _vq_body,
        grid=(_GRID,),
        in_specs=[
            pl.BlockSpec((_BLOCK, _CODE_DIM), lambda i: (i, 0)),
            pl.BlockSpec((_N_CODES, _CODE_DIM), lambda i: (0, 0)),
            pl.BlockSpec((_N_CODES, _CODE_DIM), lambda i: (0, 0)),
            pl.BlockSpec((_N_CODES, _CODE_DIM), lambda i: (0, 0)),
            pl.BlockSpec((_BLOCK, 1), lambda i: (i, 0)),
            pl.BlockSpec((1, _N_CODES), lambda i: (0, 0)),
        ],
        out_specs=[
            pl.BlockSpec((_BLOCK, _CODE_DIM), lambda i: (i, 0)),
            pl.BlockSpec((1, 1, _BLOCK), lambda i: (i, 0, 0)),
            pl.BlockSpec((1, _CODE_DIM), lambda i: (0, 0)),
        ],
        out_shape=[
            jax.ShapeDtypeStruct((_ROWS, _CODE_DIM), jnp.float32),
            jax.ShapeDtypeStruct((_GRID, 1, _BLOCK), jnp.int32),
            jax.ShapeDtypeStruct((1, _CODE_DIM), jnp.float32),
        ],
    )(zf, embedding, ehi, elo, zsq, esq)
    return zq, idx, loss


def kernel(z, embedding):
    b, n, d = z.shape
    zf = z.reshape(b * n, d)
    zsq = jnp.sum(zf ** 2, axis=-1, keepdims=True)
    esq = jnp.sum(embedding ** 2, axis=-1)[None, :]
    ehi = embedding.astype(jnp.bfloat16)
    elo = (embedding - ehi.astype(jnp.float32)).astype(jnp.bfloat16)
    zq, idx, loss = _vq(zf, embedding, ehi, elo, zsq, esq)
    vq_loss = jnp.sum(loss) * ((1.0 + _COMMITMENT) / (b * n * d))
    return zq.reshape(b, n, d), idx.reshape(b, n), vq_loss


# speedup vs baseline: 1.3665x; 1.0301x over previous
import jax
import jax.numpy as jnp
from jax.experimental import pallas as pl

_N_CODES = 512
_CODE_DIM = 32
_COMMITMENT = 0.25
_ROWS = 64 * 1024
_BLOCK = 4096
_GRID = _ROWS // _BLOCK


def _vq_body(z_ref, e_ref, ehi_ref, elo_ref, zsq_ref, esq_ref,
             zq_ref, idx_ref, loss_ref):
    i = pl.program_id(0)
    z = z_ref[...]
    e = e_ref[...]
    scores = jax.lax.dot_general(
        z.astype(jnp.bfloat16), e.astype(jnp.bfloat16), (((1,), (1,)), ((), ())),
        preferred_element_type=jnp.float32)
    base = zsq_ref[...] + esq_ref[...]
    dist = base - 2.0 * scores
    m = jnp.min(dist, axis=1, keepdims=True)
    iota = jax.lax.broadcasted_iota(jnp.int32, (_BLOCK, _N_CODES), 1)
    idx = jnp.min(jnp.where(dist == m, iota, _N_CODES), axis=1).astype(jnp.int32)
    idx_ref[0, 0, :] = idx
    onehot = (iota == idx[:, None]).astype(jnp.bfloat16)
    zq = (jax.lax.dot_general(
              onehot, ehi_ref[...], (((1,), (0,)), ((), ())),
              preferred_element_type=jnp.float32)
          + jax.lax.dot_general(
              onehot, elo_ref[...], (((1,), (0,)), ((), ())),
              preferred_element_type=jnp.float32))
    zq_ref[...] = z + (zq - z)
    diff = zq - z

    @pl.when(i == 0)
    def _init():
        loss_ref[...] = jnp.zeros_like(loss_ref)

    loss_ref[...] += jnp.sum(diff * diff, axis=0, keepdims=True)


@jax.jit
def _vq(zf, embedding, ehi, elo, zsq, esq):
    zq, idx, loss = pl.pallas_call(
        _vq_body,
        grid=(_GRID,),
        in_specs=[
            pl.BlockSpec((_BLOCK, _CODE_DIM), lambda i: (i, 0)),
            pl.BlockSpec((_N_CODES, _CODE_DIM), lambda i: (0, 0)),
            pl.BlockSpec((_N_CODES, _CODE_DIM), lambda i: (0, 0)),
            pl.BlockSpec((_N_CODES, _CODE_DIM), lambda i: (0, 0)),
            pl.BlockSpec((_BLOCK, 1), lambda i: (i, 0)),
            pl.BlockSpec((1, _N_CODES), lambda i: (0, 0)),
        ],
        out_specs=[
            pl.BlockSpec((_BLOCK, _CODE_DIM), lambda i: (i, 0)),
            pl.BlockSpec((1, 1, _BLOCK), lambda i: (i, 0, 0)),
            pl.BlockSpec((1, _CODE_DIM), lambda i: (0, 0)),
        ],
        out_shape=[
            jax.ShapeDtypeStruct((_ROWS, _CODE_DIM), jnp.float32),
            jax.ShapeDtypeStruct((_GRID, 1, _BLOCK), jnp.int32),
            jax.ShapeDtypeStruct((1, _CODE_DIM), jnp.float32),
        ],
    )(zf, embedding, ehi, elo, zsq, esq)
    return zq, idx, loss


def kernel(z, embedding):
    b, n, d = z.shape
    zf = z.reshape(b * n, d)
    zsq = jnp.sum(zf ** 2, axis=-1, keepdims=True)
    esq = jnp.sum(embedding ** 2, axis=-1)[None, :]
    ehi = embedding.astype(jnp.bfloat16)
    elo = (embedding - ehi.astype(jnp.float32)).astype(jnp.bfloat16)
    zq, idx, loss = _vq(zf, embedding, ehi, elo, zsq, esq)
    vq_loss = jnp.sum(loss) * ((1.0 + _COMMITMENT) / (b * n * d))
    return zq.reshape(b, n, d), idx.reshape(b, n), vq_loss


# speedup vs baseline: 1.3768x; 1.0075x over previous
import jax
import jax.numpy as jnp
from jax.experimental import pallas as pl

_N_CODES = 512
_CODE_DIM = 32
_COMMITMENT = 0.25
_ROWS = 64 * 1024
_BLOCK = 8192
_GRID = _ROWS // _BLOCK


def _vq_body(z_ref, e_ref, ehi_ref, elo_ref, zsq_ref, esq_ref,
             zq_ref, idx_ref, loss_ref):
    i = pl.program_id(0)
    z = z_ref[...]
    e = e_ref[...]
    scores = jax.lax.dot_general(
        z.astype(jnp.bfloat16), e.astype(jnp.bfloat16), (((1,), (1,)), ((), ())),
        preferred_element_type=jnp.float32)
    base = zsq_ref[...] + esq_ref[...]
    dist = base - 2.0 * scores
    m = jnp.min(dist, axis=1, keepdims=True)
    iota = jax.lax.broadcasted_iota(jnp.int32, (_BLOCK, _N_CODES), 1)
    idx = jnp.min(jnp.where(dist == m, iota, _N_CODES), axis=1).astype(jnp.int32)
    idx_ref[0, 0, :] = idx
    onehot = (iota == idx[:, None]).astype(jnp.bfloat16)
    zq = (jax.lax.dot_general(
              onehot, ehi_ref[...], (((1,), (0,)), ((), ())),
              preferred_element_type=jnp.float32)
          + jax.lax.dot_general(
              onehot, elo_ref[...], (((1,), (0,)), ((), ())),
              preferred_element_type=jnp.float32))
    zq_ref[...] = z + (zq - z)
    diff = zq - z

    @pl.when(i == 0)
    def _init():
        loss_ref[...] = jnp.zeros_like(loss_ref)

    loss_ref[...] += jnp.sum(diff * diff, axis=0, keepdims=True)


@jax.jit
def _vq(zf, embedding, ehi, elo, zsq, esq):
    zq, idx, loss = pl.pallas_call(
        _vq_body,
        grid=(_GRID,),
        in_specs=[
            pl.BlockSpec((_BLOCK, _CODE_DIM), lambda i: (i, 0)),
            pl.BlockSpec((_N_CODES, _CODE_DIM), lambda i: (0, 0)),
            pl.BlockSpec((_N_CODES, _CODE_DIM), lambda i: (0, 0)),
            pl.BlockSpec((_N_CODES, _CODE_DIM), lambda i: (0, 0)),
            pl.BlockSpec((_BLOCK, 1), lambda i: (i, 0)),
            pl.BlockSpec((1, _N_CODES), lambda i: (0, 0)),
        ],
        out_specs=[
            pl.BlockSpec((_BLOCK, _CODE_DIM), lambda i: (i, 0)),
            pl.BlockSpec((1, 1, _BLOCK), lambda i: (i, 0, 0)),
            pl.BlockSpec((1, _CODE_DIM), lambda i: (0, 0)),
        ],
        out_shape=[
            jax.ShapeDtypeStruct((_ROWS, _CODE_DIM), jnp.float32),
            jax.ShapeDtypeStruct((_GRID, 1, _BLOCK), jnp.int32),
            jax.ShapeDtypeStruct((1, _CODE_DIM), jnp.float32),
        ],
    )(zf, embedding, ehi, elo, zsq, esq)
    return zq, idx, loss


def kernel(z, embedding):
    b, n, d = z.shape
    zf = z.reshape(b * n, d)
    zsq = jnp.sum(zf ** 2, axis=-1, keepdims=True)
    esq = jnp.sum(embedding ** 2, axis=-1)[None, :]
    ehi = embedding.astype(jnp.bfloat16)
    elo = (embedding - ehi.astype(jnp.float32)).astype(jnp.bfloat16)
    zq, idx, loss = _vq(zf, embedding, ehi, elo, zsq, esq)
    vq_loss = jnp.sum(loss) * ((1.0 + _COMMITMENT) / (b * n * d))
    return zq.reshape(b, n, d), idx.reshape(b, n), vq_loss


# speedup vs baseline: 1.5436x; 1.1212x over previous
import jax
import jax.numpy as jnp
from jax.experimental import pallas as pl

_N_CODES = 512
_CODE_DIM = 32
_COMMITMENT = 0.25
_ROWS = 64 * 1024
_BLOCK = 8192
_GRID = _ROWS // _BLOCK


def _vq_body(z_ref, e_ref, ehi_ref, elo_ref, esq_ref,
             zq_ref, idx_ref, loss_ref):
    i = pl.program_id(0)
    z = z_ref[...]
    e = e_ref[...]
    scores = jax.lax.dot_general(
        z.astype(jnp.bfloat16), e.astype(jnp.bfloat16), (((1,), (1,)), ((), ())),
        preferred_element_type=jnp.float32)
    zsq = jnp.sum(z * z, axis=1, keepdims=True)
    base = zsq + esq_ref[...]
    dist = base - 2.0 * scores
    m = jnp.min(dist, axis=1, keepdims=True)
    iota = jax.lax.broadcasted_iota(jnp.int32, (_BLOCK, _N_CODES), 1)
    idx = jnp.min(jnp.where(dist == m, iota, _N_CODES), axis=1).astype(jnp.int32)
    idx_ref[0, 0, :] = idx
    onehot = (iota == idx[:, None]).astype(jnp.bfloat16)
    zq = (jax.lax.dot_general(
              onehot, ehi_ref[...], (((1,), (0,)), ((), ())),
              preferred_element_type=jnp.float32)
          + jax.lax.dot_general(
              onehot, elo_ref[...], (((1,), (0,)), ((), ())),
              preferred_element_type=jnp.float32))
    zq_ref[...] = z + (zq - z)
    diff = zq - z

    @pl.when(i == 0)
    def _init():
        loss_ref[...] = jnp.zeros_like(loss_ref)

    loss_ref[...] += jnp.sum(diff * diff, axis=0, keepdims=True)


@jax.jit
def _vq(zf, embedding, ehi, elo, esq):
    zq, idx, loss = pl.pallas_call(
        _vq_body,
        grid=(_GRID,),
        in_specs=[
            pl.BlockSpec((_BLOCK, _CODE_DIM), lambda i: (i, 0)),
            pl.BlockSpec((_N_CODES, _CODE_DIM), lambda i: (0, 0)),
            pl.BlockSpec((_N_CODES, _CODE_DIM), lambda i: (0, 0)),
            pl.BlockSpec((_N_CODES, _CODE_DIM), lambda i: (0, 0)),
            pl.BlockSpec((1, _N_CODES), lambda i: (0, 0)),
        ],
        out_specs=[
            pl.BlockSpec((_BLOCK, _CODE_DIM), lambda i: (i, 0)),
            pl.BlockSpec((1, 1, _BLOCK), lambda i: (i, 0, 0)),
            pl.BlockSpec((1, _CODE_DIM), lambda i: (0, 0)),
        ],
        out_shape=[
            jax.ShapeDtypeStruct((_ROWS, _CODE_DIM), jnp.float32),
            jax.ShapeDtypeStruct((_GRID, 1, _BLOCK), jnp.int32),
            jax.ShapeDtypeStruct((1, _CODE_DIM), jnp.float32),
        ],
    )(zf, embedding, ehi, elo, esq)
    return zq, idx, loss


def kernel(z, embedding):
    b, n, d = z.shape
    zf = z.reshape(b * n, d)
    esq = jnp.sum(embedding ** 2, axis=-1)[None, :]
    ehi = embedding.astype(jnp.bfloat16)
    elo = (embedding - ehi.astype(jnp.float32)).astype(jnp.bfloat16)
    zq, idx, loss = _vq(zf, embedding, ehi, elo, esq)
    vq_loss = jnp.sum(loss) * ((1.0 + _COMMITMENT) / (b * n * d))
    return zq.reshape(b, n, d), idx.reshape(b, n), vq_loss
